# baseline (speedup 1.0000x reference)
import jax
import jax.numpy as jnp
import numpy as np
from jax import lax
from jax.experimental import pallas as pl
from jax.experimental.pallas import tpu as pltpu
from jax.experimental.pallas import tpu_sc as plsc

NUM_FIELDS = 26
EMBED_DIM = 16
FIELD_SIZE = 1000
BATCH = 4096
PAIRS = [(f, g) for f in range(NUM_FIELDS - 1) for g in range(f + 1, NUM_FIELDS)]
INTER_DIM = EMBED_DIM * len(PAIRS)
_OFFS = np.arange(NUM_FIELDS, dtype=np.int32) * FIELD_SIZE

SLAB = 512
TROWS = 26112
W_COL = NUM_FIELDS * EMBED_DIM

NTILE = 41
HCOLS = NTILE * 128

NUM_SC = 2
NUM_SUBCORES = 16
NUM_WORKERS = NUM_SC * NUM_SUBCORES
SAMPLES_PER_WORKER = BATCH // NUM_WORKERS
BLOCKS_PER_WORKER = SAMPLES_PER_WORKER // 8


def _compact_body(emb_ref, w_ref, out_ref):
    x = emb_ref[...].reshape(NUM_FIELDS * EMBED_DIM, SLAB)
    xt = jnp.swapaxes(x, 0, 1)
    w = w_ref[...]
    z = jnp.zeros((SLAB, SLAB - W_COL - 1), jnp.float32)
    out_ref[...] = jnp.concatenate([xt, w, z], axis=1)


def _compact(emb_bt, w_lin):
    grid = (TROWS // SLAB,)
    return pl.pallas_call(
        _compact_body,
        grid=grid,
        in_specs=[
            pl.BlockSpec((NUM_FIELDS, EMBED_DIM, SLAB), lambda i: (0, 0, i)),
            pl.BlockSpec((SLAB, 1), lambda i: (i, 0)),
        ],
        out_specs=pl.BlockSpec((SLAB, SLAB), lambda i: (i, 0)),
        out_shape=jax.ShapeDtypeStruct((TROWS, SLAB), jnp.float32),
    )(emb_bt, w_lin)


def _sc_body(xoff_hbm, tab_hbm, h4_hbm, xoff_v, slab0_v, slab1_v, hrow0_v,
             hrow1_v, gsem0, gsem1, hsem0, hsem1):
    wid = lax.axis_index("s") * NUM_SC + lax.axis_index("c")
    base = wid * SAMPLES_PER_WORKER
    rowblk0 = wid * BLOCKS_PER_WORKER
    pltpu.sync_copy(xoff_hbm.at[pl.ds(base, SAMPLES_PER_WORKER)], xoff_v)

    zero16 = jnp.zeros((EMBED_DIM,), jnp.float32)
    for hrow in (hrow0_v, hrow1_v):
        hrow[NTILE - 1, 6, :] = zero16
        hrow[NTILE - 1, 7, :] = zero16

    def products(cs, hrow, one):
        chunk = 25
        for lo in range(0, len(PAIRS), chunk):
            def chunk_body(i, c, lo=lo):
                for p in range(lo, min(lo + chunk, len(PAIRS))):
                    f, g = PAIRS[p]
                    hrow[p // 8, p % 8, :] = (
                        cs[g, pl.ds(f * EMBED_DIM, EMBED_DIM)]
                        * cs[f, pl.ds(g * EMBED_DIM, EMBED_DIM)])
                return c
            lax.fori_loop(0, one, chunk_body, 0)

        def acc_body(i, c):
            acc = cs[0, pl.ds(W_COL, EMBED_DIM)]
            for j in range(1, NUM_FIELDS):
                acc = acc + cs[j, pl.ds(W_COL, EMBED_DIM)]
            hrow[NTILE - 1, 5, :] = acc
            return c
        lax.fori_loop(0, one, acc_body, 0)

    def hout(s):
        return h4_hbm.at[rowblk0 + s // 8, :, s % 8]

    pltpu.async_copy(tab_hbm.at[xoff_v.at[0]], slab0_v, gsem0)

    def pair_body(t, carry):
        s0 = 2 * t
        one = jnp.minimum(t + 1, 1)
        @pl.when(t >= 1)
        def _():
            pltpu.make_async_copy(hout(0), hrow0_v, hsem0).wait()
        pltpu.make_async_copy(tab_hbm.at[pl.ds(0, NUM_FIELDS)], slab0_v,
                              gsem0).wait()
        pltpu.async_copy(tab_hbm.at[xoff_v.at[s0 + 1]], slab1_v, gsem1)
        products(slab0_v, hrow0_v, one)
        pltpu.async_copy(hrow0_v, hout(s0), hsem0)
        @pl.when(t >= 1)
        def _():
            pltpu.make_async_copy(hout(0), hrow1_v, hsem1).wait()
        pltpu.make_async_copy(tab_hbm.at[pl.ds(0, NUM_FIELDS)], slab1_v,
                              gsem1).wait()
        snxt = jnp.minimum(s0 + 2, SAMPLES_PER_WORKER - 1)
        pltpu.async_copy(tab_hbm.at[xoff_v.at[snxt]], slab0_v, gsem0)
        products(slab1_v, hrow1_v, one)
        pltpu.async_copy(hrow1_v, hout(s0 + 1), hsem1)
        return carry

    lax.fori_loop(0, SAMPLES_PER_WORKER // 2, pair_body, 0)
    pltpu.make_async_copy(hout(0), hrow0_v, hsem0).wait()
    pltpu.make_async_copy(hout(0), hrow1_v, hsem1).wait()
    pltpu.make_async_copy(tab_hbm.at[pl.ds(0, NUM_FIELDS)], slab0_v,
                          gsem0).wait()


def _sc_interactions(x_off, tab):
    mesh = plsc.VectorSubcoreMesh(
        core_axis_name="c", subcore_axis_name="s",
        num_cores=NUM_SC, num_subcores=NUM_SUBCORES)
    return pl.kernel(
        _sc_body,
        out_type=jax.ShapeDtypeStruct((BATCH // 8, NTILE, 8, 8, EMBED_DIM),
                                      jnp.float32),
        mesh=mesh,
        compiler_params=pltpu.CompilerParams(use_tc_tiling_on_sc=False),
        scratch_types=[
            pltpu.VMEM((SAMPLES_PER_WORKER, NUM_FIELDS), jnp.int32),
            pltpu.VMEM((NUM_FIELDS, SLAB), jnp.float32),
            pltpu.VMEM((NUM_FIELDS, SLAB), jnp.float32),
            pltpu.VMEM((NTILE, 8, EMBED_DIM), jnp.float32),
            pltpu.VMEM((NTILE, 8, EMBED_DIM), jnp.float32),
            pltpu.SemaphoreType.DMA,
            pltpu.SemaphoreType.DMA,
            pltpu.SemaphoreType.DMA,
            pltpu.SemaphoreType.DMA,
        ],
    )(x_off, tab)


def _mlp_body(h_ref, w1_ref, b1_ref, w2_ref, b2_ref, w3_ref, b3_ref, e_ref,
              out_ref):
    x = h_ref[...]
    acc = jnp.broadcast_to(b1_ref[...], (512, 64))
    for c in range(NTILE):
        piece = x[:, c, :, :].reshape(512, 128)
        acc = acc + jnp.dot(piece, w1_ref[c],
                            preferred_element_type=jnp.float32)
    last = x[:, NTILE - 1, :, :].reshape(512, 128)
    first = jnp.dot(last, e_ref[...], preferred_element_type=jnp.float32)
    a = jnp.maximum(acc, 0.0)
    a = jnp.maximum(jnp.dot(a, w2_ref[...], preferred_element_type=jnp.float32)
                    + b2_ref[...], 0.0)
    out = jnp.dot(a, w3_ref[...], preferred_element_type=jnp.float32)
    out_ref[...] = out + b3_ref[...] + first


def _mlp(h4, W1r, b1, W2, b2, W3, b3f, e128):
    grid = (BATCH // 512,)
    return pl.pallas_call(
        _mlp_body,
        grid=grid,
        in_specs=[
            pl.BlockSpec((64, NTILE, 8, 128), lambda i: (i, 0, 0, 0)),
            pl.BlockSpec((NTILE, 128, 64), lambda i: (0, 0, 0)),
            pl.BlockSpec((1, 64), lambda i: (0, 0)),
            pl.BlockSpec((64, 32), lambda i: (0, 0)),
            pl.BlockSpec((1, 32), lambda i: (0, 0)),
            pl.BlockSpec((32, 1), lambda i: (0, 0)),
            pl.BlockSpec((1, 1), lambda i: (0, 0)),
            pl.BlockSpec((128, 1), lambda i: (0, 0)),
        ],
        out_specs=pl.BlockSpec((512, 1), lambda i: (i, 0)),
        out_shape=jax.ShapeDtypeStruct((BATCH, 1), jnp.float32),
    )(h4, W1r, b1, W2, b2, W3, b3f, e128)


def kernel(x, emb, w_lin, b_lin, W1, b1, W2, b2, W3, b3):
    x_off = x + jnp.asarray(_OFFS)[None, :]
    emb_bt = jnp.transpose(emb, (0, 2, 1))
    tab = _compact(emb_bt, w_lin)
    h4 = _sc_interactions(x_off, tab).reshape(BATCH // 8, NTILE, 8, 128)
    W1r = jnp.pad(W1, ((0, HCOLS - INTER_DIM), (0, 0))).reshape(NTILE, 128, 64)
    e128 = jnp.zeros((128, 1), jnp.float32).at[80, 0].set(1.0)
    b3f = (b3 + b_lin).reshape(1, 1)
    out = _mlp(h4, W1r, b1.reshape(1, 64), W2, b2.reshape(1, 32), W3, b3f,
               e128)
    return out[:, 0]


# speedup vs baseline: 3.3404x; 3.3404x over previous
import jax
import jax.numpy as jnp
import numpy as np
from jax import lax
from jax.experimental import pallas as pl
from jax.experimental.pallas import tpu as pltpu
from jax.experimental.pallas import tpu_sc as plsc

NUM_FIELDS = 26
EMBED_DIM = 16
FIELD_SIZE = 1000
BATCH = 4096
PAIRS = [(f, g) for f in range(NUM_FIELDS - 1) for g in range(f + 1, NUM_FIELDS)]
INTER_DIM = EMBED_DIM * len(PAIRS)
_OFFS = np.arange(NUM_FIELDS, dtype=np.int32) * FIELD_SIZE

SLAB = 512
TROWS = 26112
W_COL = NUM_FIELDS * EMBED_DIM

NTILE = 41
HCOLS = NTILE * 128

NUM_SC = 2
NUM_SUBCORES = 16
NUM_WORKERS = NUM_SC * NUM_SUBCORES
SAMPLES_PER_WORKER = BATCH // NUM_WORKERS
BLOCKS_PER_WORKER = SAMPLES_PER_WORKER // 8


def _compact_body(emb_ref, w_ref, out_ref):
    x = emb_ref[...].reshape(NUM_FIELDS * EMBED_DIM, SLAB)
    xt = jnp.swapaxes(x, 0, 1)
    w = w_ref[...]
    z = jnp.zeros((SLAB, SLAB - W_COL - 1), jnp.float32)
    out_ref[...] = jnp.concatenate([xt, w, z], axis=1)


def _compact(emb_bt, w_lin):
    grid = (TROWS // SLAB,)
    return pl.pallas_call(
        _compact_body,
        grid=grid,
        in_specs=[
            pl.BlockSpec((NUM_FIELDS, EMBED_DIM, SLAB), lambda i: (0, 0, i)),
            pl.BlockSpec((SLAB, 1), lambda i: (i, 0)),
        ],
        out_specs=pl.BlockSpec((SLAB, SLAB), lambda i: (i, 0)),
        out_shape=jax.ShapeDtypeStruct((TROWS, SLAB), jnp.float32),
    )(emb_bt, w_lin)


def _sc_body(xoff_hbm, tab_hbm, h4_hbm, xoff_v, slab0_v, slab1_v, hrow0_v,
             hrow1_v, gsem0, gsem1, hsem0, hsem1):
    wid = lax.axis_index("s") * NUM_SC + lax.axis_index("c")
    base = wid * SAMPLES_PER_WORKER
    rowblk0 = wid * BLOCKS_PER_WORKER
    pltpu.sync_copy(xoff_hbm.at[pl.ds(base, SAMPLES_PER_WORKER)], xoff_v)

    zero16 = jnp.zeros((EMBED_DIM,), jnp.float32)
    for hrow in (hrow0_v, hrow1_v):
        hrow[NTILE - 1, pl.ds(96, EMBED_DIM)] = zero16
        hrow[NTILE - 1, pl.ds(112, EMBED_DIM)] = zero16

    def products(cs, hrow):
        for p, (f, g) in enumerate(PAIRS):
            hrow[p // 8, pl.ds(EMBED_DIM * (p % 8), EMBED_DIM)] = (
                cs[g, pl.ds(f * EMBED_DIM, EMBED_DIM)]
                * cs[f, pl.ds(g * EMBED_DIM, EMBED_DIM)])
        acc = cs[0, pl.ds(W_COL, EMBED_DIM)]
        for i in range(1, NUM_FIELDS):
            acc = acc + cs[i, pl.ds(W_COL, EMBED_DIM)]
        hrow[NTILE - 1, pl.ds(80, EMBED_DIM)] = acc

    def hout(s):
        return h4_hbm.at[rowblk0 + s // 8, :, s % 8]

    pltpu.async_copy(tab_hbm.at[xoff_v.at[0]], slab0_v, gsem0)

    def pair_body(t, carry):
        s0 = 2 * t
        @pl.when(t >= 1)
        def _():
            pltpu.make_async_copy(hout(0), hrow0_v, hsem0).wait()
        pltpu.make_async_copy(tab_hbm.at[pl.ds(0, NUM_FIELDS)], slab0_v,
                              gsem0).wait()
        pltpu.async_copy(tab_hbm.at[xoff_v.at[s0 + 1]], slab1_v, gsem1)
        products(slab0_v, hrow0_v)
        pltpu.async_copy(hrow0_v, hout(s0), hsem0)
        @pl.when(t >= 1)
        def _():
            pltpu.make_async_copy(hout(0), hrow1_v, hsem1).wait()
        pltpu.make_async_copy(tab_hbm.at[pl.ds(0, NUM_FIELDS)], slab1_v,
                              gsem1).wait()
        snxt = jnp.minimum(s0 + 2, SAMPLES_PER_WORKER - 1)
        pltpu.async_copy(tab_hbm.at[xoff_v.at[snxt]], slab0_v, gsem0)
        products(slab1_v, hrow1_v)
        pltpu.async_copy(hrow1_v, hout(s0 + 1), hsem1)
        return carry

    lax.fori_loop(0, SAMPLES_PER_WORKER // 2, pair_body, 0)
    pltpu.make_async_copy(hout(0), hrow0_v, hsem0).wait()
    pltpu.make_async_copy(hout(0), hrow1_v, hsem1).wait()
    pltpu.make_async_copy(tab_hbm.at[pl.ds(0, NUM_FIELDS)], slab0_v,
                          gsem0).wait()


def _sc_interactions(x_off, tab):
    mesh = plsc.VectorSubcoreMesh(
        core_axis_name="c", subcore_axis_name="s",
        num_cores=NUM_SC, num_subcores=NUM_SUBCORES)
    return pl.kernel(
        _sc_body,
        out_type=jax.ShapeDtypeStruct((BATCH // 8, NTILE, 8, 128),
                                      jnp.float32),
        mesh=mesh,
        compiler_params=pltpu.CompilerParams(use_tc_tiling_on_sc=False),
        scratch_types=[
            pltpu.VMEM((SAMPLES_PER_WORKER, NUM_FIELDS), jnp.int32),
            pltpu.VMEM((NUM_FIELDS, SLAB), jnp.float32),
            pltpu.VMEM((NUM_FIELDS, SLAB), jnp.float32),
            pltpu.VMEM((NTILE, 128), jnp.float32),
            pltpu.VMEM((NTILE, 128), jnp.float32),
            pltpu.SemaphoreType.DMA,
            pltpu.SemaphoreType.DMA,
            pltpu.SemaphoreType.DMA,
            pltpu.SemaphoreType.DMA,
        ],
    )(x_off, tab)


def _mlp_body(h_ref, w1_ref, b1_ref, w2_ref, b2_ref, w3_ref, b3_ref, e_ref,
              out_ref):
    x = h_ref[...]
    acc = jnp.broadcast_to(b1_ref[...], (512, 64))
    for c in range(NTILE):
        piece = x[:, c, :, :].reshape(512, 128)
        acc = acc + jnp.dot(piece, w1_ref[c],
                            preferred_element_type=jnp.float32)
    last = x[:, NTILE - 1, :, :].reshape(512, 128)
    first = jnp.dot(last, e_ref[...], preferred_element_type=jnp.float32)
    a = jnp.maximum(acc, 0.0)
    a = jnp.maximum(jnp.dot(a, w2_ref[...], preferred_element_type=jnp.float32)
                    + b2_ref[...], 0.0)
    out = jnp.dot(a, w3_ref[...], preferred_element_type=jnp.float32)
    out_ref[...] = out + b3_ref[...] + first


def _mlp(h4, W1r, b1, W2, b2, W3, b3f, e128):
    grid = (BATCH // 512,)
    return pl.pallas_call(
        _mlp_body,
        grid=grid,
        in_specs=[
            pl.BlockSpec((64, NTILE, 8, 128), lambda i: (i, 0, 0, 0)),
            pl.BlockSpec((NTILE, 128, 64), lambda i: (0, 0, 0)),
            pl.BlockSpec((1, 64), lambda i: (0, 0)),
            pl.BlockSpec((64, 32), lambda i: (0, 0)),
            pl.BlockSpec((1, 32), lambda i: (0, 0)),
            pl.BlockSpec((32, 1), lambda i: (0, 0)),
            pl.BlockSpec((1, 1), lambda i: (0, 0)),
            pl.BlockSpec((128, 1), lambda i: (0, 0)),
        ],
        out_specs=pl.BlockSpec((512, 1), lambda i: (i, 0)),
        out_shape=jax.ShapeDtypeStruct((BATCH, 1), jnp.float32),
    )(h4, W1r, b1, W2, b2, W3, b3f, e128)


def kernel(x, emb, w_lin, b_lin, W1, b1, W2, b2, W3, b3):
    x_off = x + jnp.asarray(_OFFS)[None, :]
    emb_bt = jnp.transpose(emb, (0, 2, 1))
    tab = _compact(emb_bt, w_lin)
    h4 = _sc_interactions(x_off, tab)
    W1r = jnp.pad(W1, ((0, HCOLS - INTER_DIM), (0, 0))).reshape(NTILE, 128, 64)
    e128 = jnp.zeros((128, 1), jnp.float32).at[80, 0].set(1.0)
    b3f = (b3 + b_lin).reshape(1, 1)
    out = _mlp(h4, W1r, b1.reshape(1, 64), W2, b2.reshape(1, 32), W3, b3f,
               e128)
    return out[:, 0]


# speedup vs baseline: 3.9110x; 1.1708x over previous
import jax
import jax.numpy as jnp
import numpy as np
from jax import lax
from jax.experimental import pallas as pl
from jax.experimental.pallas import tpu as pltpu
from jax.experimental.pallas import tpu_sc as plsc

NUM_FIELDS = 26
EMBED_DIM = 16
FIELD_SIZE = 1000
BATCH = 4096
PAIRS = [(f, g) for f in range(NUM_FIELDS - 1) for g in range(f + 1, NUM_FIELDS)]
INTER_DIM = EMBED_DIM * len(PAIRS)
_OFFS = np.arange(NUM_FIELDS, dtype=np.int32) * FIELD_SIZE

SLAB = 512
TROWS = 26112
W_COL = NUM_FIELDS * EMBED_DIM

NTILE = 41
HCOLS = NTILE * 128

NUM_SC = 2
NUM_SUBCORES = 16
NUM_WORKERS = NUM_SC * NUM_SUBCORES
SAMPLES_PER_WORKER = BATCH // NUM_WORKERS
BLOCKS_PER_WORKER = SAMPLES_PER_WORKER // 8


def _compact_body(emb_ref, w_ref, out_ref):
    x = emb_ref[...].reshape(NUM_FIELDS * EMBED_DIM, SLAB)
    xt = jnp.swapaxes(x, 0, 1)
    w = w_ref[...]
    z = jnp.zeros((SLAB, SLAB - W_COL - 1), jnp.float32)
    slab = jnp.concatenate([xt, w, z], axis=1)
    out_ref[...] = slab.reshape(SLAB * 4, 128)


def _compact(emb_bt, w_lin):
    grid = (TROWS // SLAB,)
    return pl.pallas_call(
        _compact_body,
        grid=grid,
        in_specs=[
            pl.BlockSpec((NUM_FIELDS, EMBED_DIM, SLAB), lambda i: (0, 0, i)),
            pl.BlockSpec((SLAB, 1), lambda i: (i, 0)),
        ],
        out_specs=pl.BlockSpec((SLAB * 4, 128), lambda i: (i, 0)),
        out_shape=jax.ShapeDtypeStruct((TROWS * 4, 128), jnp.float32),
    )(emb_bt, w_lin).reshape(TROWS, SLAB)


def _sc_body(xoff_hbm, tab_hbm, h4_hbm, xoff_v,
             slab0_v, slab1_v, slab2_v, slab3_v,
             hrow0_v, hrow1_v, hrow2_v, hrow3_v,
             gsem0, gsem1, gsem2, gsem3, hsem0, hsem1, hsem2, hsem3):
    wid = lax.axis_index("s") * NUM_SC + lax.axis_index("c")
    base = wid * SAMPLES_PER_WORKER
    rowblk0 = wid * BLOCKS_PER_WORKER
    pltpu.sync_copy(xoff_hbm.at[pl.ds(base, SAMPLES_PER_WORKER)], xoff_v)

    slabs = (slab0_v, slab1_v, slab2_v, slab3_v)
    hrows = (hrow0_v, hrow1_v, hrow2_v, hrow3_v)
    gsems = (gsem0, gsem1, gsem2, gsem3)
    hsems = (hsem0, hsem1, hsem2, hsem3)

    zero16 = jnp.zeros((EMBED_DIM,), jnp.float32)
    for hrow in hrows:
        hrow[NTILE - 1, pl.ds(96, EMBED_DIM)] = zero16
        hrow[NTILE - 1, pl.ds(112, EMBED_DIM)] = zero16

    def products(cs, hrow):
        for p, (f, g) in enumerate(PAIRS):
            if p and p % 25 == 0:
                pltpu.trace_value("chunk", jnp.int32(p))
            hrow[p // 8, pl.ds(EMBED_DIM * (p % 8), EMBED_DIM)] = (
                cs[g, pl.ds(f * EMBED_DIM, EMBED_DIM)]
                * cs[f, pl.ds(g * EMBED_DIM, EMBED_DIM)])
        acc = cs[0, pl.ds(W_COL, EMBED_DIM)]
        for i in range(1, NUM_FIELDS):
            acc = acc + cs[i, pl.ds(W_COL, EMBED_DIM)]
        hrow[NTILE - 1, pl.ds(80, EMBED_DIM)] = acc

    def hout(s):
        return h4_hbm.at[rowblk0 + s // 8, :, s % 8]

    pltpu.async_copy(tab_hbm.at[xoff_v.at[0]], slab0_v, gsem0)
    pltpu.async_copy(tab_hbm.at[xoff_v.at[1]], slab1_v, gsem1)

    def quad_body(t, carry):
        s0 = 4 * t
        for j in range(4):
            s = s0 + j
            @pl.when(t >= 1)
            def _(j=j):
                pltpu.make_async_copy(hout(0), hrows[j], hsems[j]).wait()
            pltpu.make_async_copy(tab_hbm.at[pl.ds(0, NUM_FIELDS)], slabs[j],
                                  gsems[j]).wait()
            snxt = jnp.minimum(s + 2, SAMPLES_PER_WORKER - 1)
            pltpu.async_copy(tab_hbm.at[xoff_v.at[snxt]],
                             slabs[(j + 2) % 4], gsems[(j + 2) % 4])
            products(slabs[j], hrows[j])
            pltpu.async_copy(hrows[j], hout(s), hsems[j])
        return carry

    lax.fori_loop(0, SAMPLES_PER_WORKER // 4, quad_body, 0)
    for j in range(4):
        pltpu.make_async_copy(hout(0), hrows[j], hsems[j]).wait()
    pltpu.make_async_copy(tab_hbm.at[pl.ds(0, NUM_FIELDS)], slab0_v,
                          gsem0).wait()
    pltpu.make_async_copy(tab_hbm.at[pl.ds(0, NUM_FIELDS)], slab1_v,
                          gsem1).wait()


def _sc_interactions(x_off, tab):
    mesh = plsc.VectorSubcoreMesh(
        core_axis_name="c", subcore_axis_name="s",
        num_cores=NUM_SC, num_subcores=NUM_SUBCORES)
    return pl.kernel(
        _sc_body,
        out_type=jax.ShapeDtypeStruct((BATCH // 8, NTILE, 8, 128),
                                      jnp.float32),
        mesh=mesh,
        compiler_params=pltpu.CompilerParams(use_tc_tiling_on_sc=False),
        scratch_types=[
            pltpu.VMEM((SAMPLES_PER_WORKER, NUM_FIELDS), jnp.int32),
            pltpu.VMEM((NUM_FIELDS, SLAB), jnp.float32),
            pltpu.VMEM((NUM_FIELDS, SLAB), jnp.float32),
            pltpu.VMEM((NUM_FIELDS, SLAB), jnp.float32),
            pltpu.VMEM((NUM_FIELDS, SLAB), jnp.float32),
            pltpu.VMEM((NTILE, 128), jnp.float32),
            pltpu.VMEM((NTILE, 128), jnp.float32),
            pltpu.VMEM((NTILE, 128), jnp.float32),
            pltpu.VMEM((NTILE, 128), jnp.float32),
            pltpu.SemaphoreType.DMA,
            pltpu.SemaphoreType.DMA,
            pltpu.SemaphoreType.DMA,
            pltpu.SemaphoreType.DMA,
            pltpu.SemaphoreType.DMA,
            pltpu.SemaphoreType.DMA,
            pltpu.SemaphoreType.DMA,
            pltpu.SemaphoreType.DMA,
        ],
    )(x_off, tab)


def _mlp_body(h_ref, w1_ref, b1_ref, w2_ref, b2_ref, w3_ref, b3_ref, e_ref,
              out_ref):
    x = h_ref[...]
    acc = jnp.broadcast_to(b1_ref[...], (512, 64))
    for c in range(NTILE):
        piece = x[:, c, :, :].reshape(512, 128)
        acc = acc + jnp.dot(piece, w1_ref[c],
                            preferred_element_type=jnp.float32)
    last = x[:, NTILE - 1, :, :].reshape(512, 128)
    first = jnp.dot(last, e_ref[...], preferred_element_type=jnp.float32)
    a = jnp.maximum(acc, 0.0)
    a = jnp.maximum(jnp.dot(a, w2_ref[...], preferred_element_type=jnp.float32)
                    + b2_ref[...], 0.0)
    out = jnp.dot(a, w3_ref[...], preferred_element_type=jnp.float32)
    out_ref[...] = out + b3_ref[...] + first


def _mlp(h4, W1r, b1, W2, b2, W3, b3f, e128):
    grid = (BATCH // 512,)
    return pl.pallas_call(
        _mlp_body,
        grid=grid,
        in_specs=[
            pl.BlockSpec((64, NTILE, 8, 128), lambda i: (i, 0, 0, 0)),
            pl.BlockSpec((NTILE, 128, 64), lambda i: (0, 0, 0)),
            pl.BlockSpec((1, 64), lambda i: (0, 0)),
            pl.BlockSpec((64, 32), lambda i: (0, 0)),
            pl.BlockSpec((1, 32), lambda i: (0, 0)),
            pl.BlockSpec((32, 1), lambda i: (0, 0)),
            pl.BlockSpec((1, 1), lambda i: (0, 0)),
            pl.BlockSpec((128, 1), lambda i: (0, 0)),
        ],
        out_specs=pl.BlockSpec((512, 1), lambda i: (i, 0)),
        out_shape=jax.ShapeDtypeStruct((BATCH, 1), jnp.float32),
    )(h4, W1r, b1, W2, b2, W3, b3f, e128)


def kernel(x, emb, w_lin, b_lin, W1, b1, W2, b2, W3, b3):
    x_off = x + jnp.asarray(_OFFS)[None, :]
    emb_bt = jnp.transpose(emb, (0, 2, 1))
    tab = _compact(emb_bt, w_lin)
    h4 = _sc_interactions(x_off, tab)
    W1r = jnp.pad(W1, ((0, HCOLS - INTER_DIM), (0, 0))).reshape(NTILE, 128, 64)
    e128 = jnp.zeros((128, 1), jnp.float32).at[80, 0].set(1.0)
    b3f = (b3 + b_lin).reshape(1, 1)
    out = _mlp(h4, W1r, b1.reshape(1, 64), W2, b2.reshape(1, 32), W3, b3f,
               e128)
    return out[:, 0]


# speedup vs baseline: 4.1274x; 1.0553x over previous
import jax
import jax.numpy as jnp
import numpy as np
from jax import lax
from jax.experimental import pallas as pl
from jax.experimental.pallas import tpu as pltpu
from jax.experimental.pallas import tpu_sc as plsc

NUM_FIELDS = 26
EMBED_DIM = 16
FIELD_SIZE = 1000
BATCH = 4096
PAIRS = [(f, g) for f in range(NUM_FIELDS - 1) for g in range(f + 1, NUM_FIELDS)]
INTER_DIM = EMBED_DIM * len(PAIRS)
_OFFS = np.arange(NUM_FIELDS, dtype=np.int32) * FIELD_SIZE

SLAB = 512
TROWS = 26112
W_COL = NUM_FIELDS * EMBED_DIM

NTILE = 41
HCOLS = NTILE * 128

NUM_SC = 2
NUM_SUBCORES = 16
NUM_WORKERS = NUM_SC * NUM_SUBCORES
BATCH_H = BATCH // 2
SAMPLES_PER_WORKER = BATCH_H // NUM_WORKERS
BLOCKS_PER_WORKER = SAMPLES_PER_WORKER // 8


def _compact_body(emb_ref, w_ref, out_ref):
    x = emb_ref[...].reshape(NUM_FIELDS * EMBED_DIM, SLAB)
    xt = jnp.swapaxes(x, 0, 1)
    w = w_ref[...]
    z = jnp.zeros((SLAB, SLAB - W_COL - 1), jnp.float32)
    slab = jnp.concatenate([xt, w, z], axis=1)
    out_ref[...] = slab.reshape(SLAB * 4, 128)


def _compact(emb_bt, w_lin):
    grid = (TROWS // SLAB,)
    return pl.pallas_call(
        _compact_body,
        grid=grid,
        in_specs=[
            pl.BlockSpec((NUM_FIELDS, EMBED_DIM, SLAB), lambda i: (0, 0, i)),
            pl.BlockSpec((SLAB, 1), lambda i: (i, 0)),
        ],
        out_specs=pl.BlockSpec((SLAB * 4, 128), lambda i: (i, 0)),
        out_shape=jax.ShapeDtypeStruct((TROWS * 4, 128), jnp.float32),
    )(emb_bt, w_lin).reshape(TROWS, SLAB)


def _sc_body(xoff_hbm, tab_hbm, h4_hbm, xoff_v,
             slab0_v, slab1_v, slab2_v, slab3_v,
             hrow0_v, hrow1_v, hrow2_v, hrow3_v,
             gsem0, gsem1, gsem2, gsem3, hsem0, hsem1, hsem2, hsem3):
    wid = lax.axis_index("s") * NUM_SC + lax.axis_index("c")
    base = wid * SAMPLES_PER_WORKER
    rowblk0 = wid * BLOCKS_PER_WORKER
    pltpu.sync_copy(xoff_hbm.at[pl.ds(base, SAMPLES_PER_WORKER)], xoff_v)

    slabs = (slab0_v, slab1_v, slab2_v, slab3_v)
    hrows = (hrow0_v, hrow1_v, hrow2_v, hrow3_v)
    gsems = (gsem0, gsem1, gsem2, gsem3)
    hsems = (hsem0, hsem1, hsem2, hsem3)

    zero16 = jnp.zeros((EMBED_DIM,), jnp.float32)
    for hrow in hrows:
        hrow[NTILE - 1, pl.ds(96, EMBED_DIM)] = zero16
        hrow[NTILE - 1, pl.ds(112, EMBED_DIM)] = zero16

    def products(cs, hrow):
        for p, (f, g) in enumerate(PAIRS):
            if p and p % 25 == 0:
                pltpu.trace_value("chunk", jnp.int32(p))
            hrow[p // 8, pl.ds(EMBED_DIM * (p % 8), EMBED_DIM)] = (
                cs[g, pl.ds(f * EMBED_DIM, EMBED_DIM)]
                * cs[f, pl.ds(g * EMBED_DIM, EMBED_DIM)])
        acc = cs[0, pl.ds(W_COL, EMBED_DIM)]
        for i in range(1, NUM_FIELDS):
            acc = acc + cs[i, pl.ds(W_COL, EMBED_DIM)]
        hrow[NTILE - 1, pl.ds(80, EMBED_DIM)] = acc

    def hout(s):
        return h4_hbm.at[rowblk0 + s // 8, :, s % 8]

    pltpu.async_copy(tab_hbm.at[xoff_v.at[0]], slab0_v, gsem0)
    pltpu.async_copy(tab_hbm.at[xoff_v.at[1]], slab1_v, gsem1)

    def quad_body(t, carry):
        s0 = 4 * t
        for j in range(4):
            s = s0 + j
            @pl.when(t >= 1)
            def _(j=j):
                pltpu.make_async_copy(hout(0), hrows[j], hsems[j]).wait()
            pltpu.make_async_copy(tab_hbm.at[pl.ds(0, NUM_FIELDS)], slabs[j],
                                  gsems[j]).wait()
            snxt = jnp.minimum(s + 2, SAMPLES_PER_WORKER - 1)
            pltpu.async_copy(tab_hbm.at[xoff_v.at[snxt]],
                             slabs[(j + 2) % 4], gsems[(j + 2) % 4])
            products(slabs[j], hrows[j])
            pltpu.async_copy(hrows[j], hout(s), hsems[j])
        return carry

    lax.fori_loop(0, SAMPLES_PER_WORKER // 4, quad_body, 0)
    for j in range(4):
        pltpu.make_async_copy(hout(0), hrows[j], hsems[j]).wait()
    pltpu.make_async_copy(tab_hbm.at[pl.ds(0, NUM_FIELDS)], slab0_v,
                          gsem0).wait()
    pltpu.make_async_copy(tab_hbm.at[pl.ds(0, NUM_FIELDS)], slab1_v,
                          gsem1).wait()


def _sc_interactions(x_off, tab):
    mesh = plsc.VectorSubcoreMesh(
        core_axis_name="c", subcore_axis_name="s",
        num_cores=NUM_SC, num_subcores=NUM_SUBCORES)
    return pl.kernel(
        _sc_body,
        out_type=jax.ShapeDtypeStruct((BATCH_H // 8, NTILE, 8, 128),
                                      jnp.float32),
        mesh=mesh,
        compiler_params=pltpu.CompilerParams(use_tc_tiling_on_sc=False),
        scratch_types=[
            pltpu.VMEM((SAMPLES_PER_WORKER, NUM_FIELDS), jnp.int32),
            pltpu.VMEM((NUM_FIELDS, SLAB), jnp.float32),
            pltpu.VMEM((NUM_FIELDS, SLAB), jnp.float32),
            pltpu.VMEM((NUM_FIELDS, SLAB), jnp.float32),
            pltpu.VMEM((NUM_FIELDS, SLAB), jnp.float32),
            pltpu.VMEM((NTILE, 128), jnp.float32),
            pltpu.VMEM((NTILE, 128), jnp.float32),
            pltpu.VMEM((NTILE, 128), jnp.float32),
            pltpu.VMEM((NTILE, 128), jnp.float32),
            pltpu.SemaphoreType.DMA,
            pltpu.SemaphoreType.DMA,
            pltpu.SemaphoreType.DMA,
            pltpu.SemaphoreType.DMA,
            pltpu.SemaphoreType.DMA,
            pltpu.SemaphoreType.DMA,
            pltpu.SemaphoreType.DMA,
            pltpu.SemaphoreType.DMA,
        ],
    )(x_off, tab)


def _mlp_body(h_ref, w1_ref, b1_ref, w2_ref, b2_ref, w3_ref, b3_ref, e_ref,
              out_ref):
    x = h_ref[...]
    acc = jnp.broadcast_to(b1_ref[...], (512, 64))
    for c in range(NTILE):
        piece = x[:, c, :, :].reshape(512, 128)
        acc = acc + jnp.dot(piece, w1_ref[c],
                            preferred_element_type=jnp.float32)
    last = x[:, NTILE - 1, :, :].reshape(512, 128)
    first = jnp.dot(last, e_ref[...], preferred_element_type=jnp.float32)
    a = jnp.maximum(acc, 0.0)
    a = jnp.maximum(jnp.dot(a, w2_ref[...], preferred_element_type=jnp.float32)
                    + b2_ref[...], 0.0)
    out = jnp.dot(a, w3_ref[...], preferred_element_type=jnp.float32)
    out_ref[...] = out + b3_ref[...] + first


def _mlp(h4, W1r, b1, W2, b2, W3, b3f, e128):
    grid = (BATCH_H // 512,)
    return pl.pallas_call(
        _mlp_body,
        grid=grid,
        in_specs=[
            pl.BlockSpec((64, NTILE, 8, 128), lambda i: (i, 0, 0, 0)),
            pl.BlockSpec((NTILE, 128, 64), lambda i: (0, 0, 0)),
            pl.BlockSpec((1, 64), lambda i: (0, 0)),
            pl.BlockSpec((64, 32), lambda i: (0, 0)),
            pl.BlockSpec((1, 32), lambda i: (0, 0)),
            pl.BlockSpec((32, 1), lambda i: (0, 0)),
            pl.BlockSpec((1, 1), lambda i: (0, 0)),
            pl.BlockSpec((128, 1), lambda i: (0, 0)),
        ],
        out_specs=pl.BlockSpec((512, 1), lambda i: (i, 0)),
        out_shape=jax.ShapeDtypeStruct((BATCH_H, 1), jnp.float32),
    )(h4, W1r, b1, W2, b2, W3, b3f, e128)


def kernel(x, emb, w_lin, b_lin, W1, b1, W2, b2, W3, b3):
    x_off = x + jnp.asarray(_OFFS)[None, :]
    emb_bt = jnp.transpose(emb, (0, 2, 1))
    tab = _compact(emb_bt, w_lin)
    W1r = jnp.pad(W1, ((0, HCOLS - INTER_DIM), (0, 0))).reshape(NTILE, 128, 64)
    e128 = jnp.zeros((128, 1), jnp.float32).at[80, 0].set(1.0)
    b3f = (b3 + b_lin).reshape(1, 1)
    outs = []
    for half in range(2):
        h4 = _sc_interactions(x_off[half * BATCH_H:(half + 1) * BATCH_H], tab)
        outs.append(_mlp(h4, W1r, b1.reshape(1, 64), W2, b2.reshape(1, 32),
                         W3, b3f, e128))
    return jnp.concatenate(outs, axis=0)[:, 0]
